# baseline (speedup 1.0000x reference)
import functools

import jax
import jax.numpy as jnp
from jax import lax
from jax.experimental import pallas as pl
from jax.experimental.pallas import tpu as pltpu
from jax.experimental.pallas import tpu_sc as plsc

VOCAB = 1000000
EMB = 16
HID = 32
BATCH = 16384

NC = 2
NS = 16
NW = NC * NS
NIDX = 4 * BATCH
B_PER_W = NIDX // NW
CH = 64
NCHK = B_PER_W // CH

NGRP = VOCAB // 8
PK = 128 // EMB
QTOT = NIDX // PK
QC = BATCH // PK
Q_PER_W = B_PER_W // PK
Q_PER_CH = CH // PK

DP_CH = 1000

_mesh = plsc.VectorSubcoreMesh(
    core_axis_name="c", subcore_axis_name="s", num_cores=NC, num_subcores=NS
)
_sc_params = pltpu.CompilerParams(
    use_tc_tiling_on_sc=True, needs_layout_passes=False
)


def _depad_body(t_ref, o_ref):
    x = t_ref[...].reshape(DP_CH, 8, EMB)
    o_ref[...] = jnp.concatenate([x[:, s, :] for s in range(8)], axis=-1)


def _depad(table):
    return pl.pallas_call(
        _depad_body,
        grid=(NGRP // DP_CH,),
        in_specs=[pl.BlockSpec((DP_CH * 8, EMB), lambda i: (i, 0))],
        out_specs=pl.BlockSpec((DP_CH, 128), lambda i: (i, 0)),
        out_shape=jax.ShapeDtypeStruct((NGRP, 128), jnp.float32),
    )(table)


@functools.partial(
    pl.kernel,
    mesh=_mesh,
    compiler_params=_sc_params,
    out_type=jax.ShapeDtypeStruct((QTOT, PK * EMB), jnp.float32),
    scratch_types=[
        pltpu.VMEM((NCHK, 128), jnp.int32),
        pltpu.VMEM((CH, 128), jnp.float32),
        pltpu.VMEM((Q_PER_CH, PK * EMB), jnp.float32),
        pltpu.SemaphoreType.DMA,
    ],
)
def _gather_kernel(qr_hbm, lin_hbm, out_hbm, qr_v, slab_v, pk_v, sem):
    wid = lax.axis_index("s") * NC + lax.axis_index("c")
    pltpu.sync_copy(qr_hbm.at[wid], qr_v)
    obase = wid * Q_PER_W

    def chunk_body(c, carry):
        pltpu.async_copy(lin_hbm.at[qr_v.at[c, pl.ds(0, CH)]], slab_v, sem).wait()
        for g in range(CH // 16):
            qvec = lax.iota(jnp.int32, 16) + g * 16
            rvec = qr_v[c, pl.ds(CH + g * 16, 16)]
            prow = lax.shift_right_logical(lax.iota(jnp.int32, 16), 3) + g * 2
            pcol = (lax.iota(jnp.int32, 16) & 7) * EMB
            for d in range(EMB):
                col = plsc.load_gather(slab_v, [qvec, rvec * EMB + d])
                plsc.store_scatter(pk_v, [prow, pcol + d], col)
        pltpu.sync_copy(pk_v, out_hbm.at[pl.ds(obase + c * Q_PER_CH, Q_PER_CH)])
        return carry

    lax.fori_loop(0, NCHK, chunk_body, 0)


BQ = 512
GRID = QC // BQ


def _mlp_body(e_ref, w_ref, b_ref, ws_ref, bs_ref, o_ref):
    acc = jnp.zeros((BQ, PK * HID), jnp.float32)
    for c in range(4):
        h = jnp.dot(e_ref[c], w_ref[c], preferred_element_type=jnp.float32) + b_ref[c]
        acc = acc + jnp.maximum(h, 0.0)
    o_ref[...] = (
        jnp.dot(acc, ws_ref[...], preferred_element_type=jnp.float32) + bs_ref[...]
    )


def kernel(x, table, w1, b1, w2, b2, w3, b3, w4, b4, ws, bs):
    idx = jnp.stack([x[:, 0], x[:, 1], x[:, 3], x[:, 4]], axis=0)
    idx = idx.reshape(NW, NCHK, CH)
    qr = jnp.concatenate([idx >> 3, idx & 7], axis=-1)

    lin = _depad(table)
    rows_pk = _gather_kernel(qr, lin)
    e_pk = rows_pk.reshape(4, QC, PK * EMB)

    eye = jnp.eye(PK, dtype=jnp.float32)
    w_bd = jnp.stack(
        [jnp.kron(eye, w) for w in (w1, w2, w3, w4)], axis=0
    )
    b_bd = jnp.stack(
        [jnp.tile(b, PK) for b in (b1, b2, b3, b4)], axis=0
    )[:, None, :]
    ws_bd = jnp.kron(eye, ws)
    bs_bd = jnp.tile(bs, PK)[None, :]

    out = pl.pallas_call(
        _mlp_body,
        grid=(GRID,),
        in_specs=[
            pl.BlockSpec((4, BQ, PK * EMB), lambda i: (0, i, 0)),
            pl.BlockSpec((4, PK * EMB, PK * HID), lambda i: (0, 0, 0)),
            pl.BlockSpec((4, 1, PK * HID), lambda i: (0, 0, 0)),
            pl.BlockSpec((PK * HID, PK * EMB), lambda i: (0, 0)),
            pl.BlockSpec((1, PK * EMB), lambda i: (0, 0)),
        ],
        out_specs=pl.BlockSpec((BQ, PK * EMB), lambda i: (i, 0)),
        out_shape=jax.ShapeDtypeStruct((QC, PK * EMB), jnp.float32),
    )(e_pk, w_bd, b_bd, ws_bd, bs_bd)
    return out.reshape(BATCH, EMB)


# speedup vs baseline: 1.1878x; 1.1878x over previous
import functools

import jax
import jax.numpy as jnp
from jax import lax
from jax.experimental import pallas as pl
from jax.experimental.pallas import tpu as pltpu
from jax.experimental.pallas import tpu_sc as plsc

VOCAB = 1000000
EMB = 16
HID = 32
BATCH = 16384

NC = 2
NS = 16
NW = NC * NS
NIDX = 4 * BATCH
B_PER_W = NIDX // NW
CH = 64
NCHK = B_PER_W // CH

NGRP = VOCAB // 8
PK = 128 // EMB
QTOT = NIDX // PK
QC = BATCH // PK
Q_PER_W = B_PER_W // PK
Q_PER_CH = CH // PK

DP_CH = 1000

_mesh = plsc.VectorSubcoreMesh(
    core_axis_name="c", subcore_axis_name="s", num_cores=NC, num_subcores=NS
)
_sc_params = pltpu.CompilerParams(
    use_tc_tiling_on_sc=True, needs_layout_passes=False
)


def _depad_body(t_ref, o_ref):
    x = t_ref[...].reshape(DP_CH, 8, EMB)
    o_ref[...] = jnp.concatenate([x[:, s, :] for s in range(8)], axis=-1)


def _depad(table):
    return pl.pallas_call(
        _depad_body,
        grid=(NGRP // DP_CH,),
        in_specs=[pl.BlockSpec((DP_CH * 8, EMB), lambda i: (i, 0))],
        out_specs=pl.BlockSpec((DP_CH, 128), lambda i: (i, 0)),
        out_shape=jax.ShapeDtypeStruct((NGRP, 128), jnp.float32),
    )(table)


@functools.partial(
    pl.kernel,
    mesh=_mesh,
    compiler_params=_sc_params,
    out_type=jax.ShapeDtypeStruct((QTOT, PK * EMB), jnp.float32),
    scratch_types=[
        pltpu.VMEM((NCHK, 128), jnp.int32),
        pltpu.VMEM((CH, 128), jnp.float32),
        pltpu.VMEM((Q_PER_CH, PK * EMB), jnp.float32),
        pltpu.SemaphoreType.DMA,
    ],
)
def _gather_kernel(qr_hbm, lin_hbm, out_hbm, qr_v, slab_v, pk_v, sem):
    wid = lax.axis_index("s") * NC + lax.axis_index("c")
    pltpu.sync_copy(qr_hbm.at[wid], qr_v)
    obase = wid * Q_PER_W

    def chunk_body(c, carry):
        pltpu.async_copy(lin_hbm.at[qr_v.at[c, pl.ds(0, CH)]], slab_v, sem).wait()
        for g in range(CH // 16):
            qvec = lax.iota(jnp.int32, 16) + g * 16
            rvec = qr_v[c, pl.ds(CH + g * 16, 16)]
            prow = lax.shift_right_logical(lax.iota(jnp.int32, 16), 3) + g * 2
            pcol = (lax.iota(jnp.int32, 16) & 7) * EMB
            for d in range(EMB):
                col = plsc.load_gather(slab_v, [qvec, rvec * EMB + d])
                plsc.store_scatter(pk_v, [prow, pcol + d], col)
        pltpu.sync_copy(pk_v, out_hbm.at[pl.ds(obase + c * Q_PER_CH, Q_PER_CH)])
        return carry

    lax.fori_loop(0, NCHK, chunk_body, 0)


BQ = 512
GRID = QC // BQ


def _mlp_body(e_ref, w_ref, b_ref, ws_ref, bs_ref, o_ref):
    acc = jnp.zeros((BQ, PK * HID), jnp.float32)
    for c in range(4):
        h = jnp.dot(e_ref[c], w_ref[c], preferred_element_type=jnp.float32) + b_ref[c]
        acc = acc + jnp.maximum(h, 0.0)
    o_ref[...] = (
        jnp.dot(acc, ws_ref[...], preferred_element_type=jnp.float32) + bs_ref[...]
    )


def kernel(x, table, w1, b1, w2, b2, w3, b3, w4, b4, ws, bs):
    idx = jnp.stack([x[:, 0], x[:, 1], x[:, 3], x[:, 4]], axis=0)
    idx = idx.reshape(NW, NCHK, CH)
    qr = jnp.concatenate([idx >> 3, idx & 7], axis=-1)

    lin = table.reshape(NGRP, 128)
    rows_pk = _gather_kernel(qr, lin)
    e_pk = rows_pk.reshape(4, QC, PK * EMB)

    eye = jnp.eye(PK, dtype=jnp.float32)
    w_bd = jnp.stack(
        [jnp.kron(eye, w) for w in (w1, w2, w3, w4)], axis=0
    )
    b_bd = jnp.stack(
        [jnp.tile(b, PK) for b in (b1, b2, b3, b4)], axis=0
    )[:, None, :]
    ws_bd = jnp.kron(eye, ws)
    bs_bd = jnp.tile(bs, PK)[None, :]

    out = pl.pallas_call(
        _mlp_body,
        grid=(GRID,),
        in_specs=[
            pl.BlockSpec((4, BQ, PK * EMB), lambda i: (0, i, 0)),
            pl.BlockSpec((4, PK * EMB, PK * HID), lambda i: (0, 0, 0)),
            pl.BlockSpec((4, 1, PK * HID), lambda i: (0, 0, 0)),
            pl.BlockSpec((PK * HID, PK * EMB), lambda i: (0, 0)),
            pl.BlockSpec((1, PK * EMB), lambda i: (0, 0)),
        ],
        out_specs=pl.BlockSpec((BQ, PK * EMB), lambda i: (i, 0)),
        out_shape=jax.ShapeDtypeStruct((QC, PK * EMB), jnp.float32),
    )(e_pk, w_bd, b_bd, ws_bd, bs_bd)
    return out.reshape(BATCH, EMB)


# speedup vs baseline: 1.3129x; 1.1054x over previous
import functools

import jax
import jax.numpy as jnp
from jax import lax
from jax.experimental import pallas as pl
from jax.experimental.pallas import tpu as pltpu
from jax.experimental.pallas import tpu_sc as plsc

VOCAB = 1000000
EMB = 16
HID = 32
BATCH = 16384

NC = 2
NS = 16
NW = NC * NS
NIDX = 4 * BATCH
B_PER_W = NIDX // NW
CHUNK = 128
NCHUNK = B_PER_W // CHUNK

PK = 128 // EMB
QTOT = NIDX // PK
QC = BATCH // PK


def _make_gather():
    mesh = plsc.VectorSubcoreMesh(
        core_axis_name="c", subcore_axis_name="s", num_cores=NC, num_subcores=NS
    )

    @functools.partial(
        pl.kernel,
        mesh=mesh,
        compiler_params=pltpu.CompilerParams(use_tc_tiling_on_sc=False),
        out_type=jax.ShapeDtypeStruct((NIDX, EMB), jnp.float32),
        scratch_types=[
            pltpu.VMEM((NCHUNK, CHUNK), jnp.int32),
            pltpu.VMEM((B_PER_W, EMB), jnp.float32),
            pltpu.SemaphoreType.DMA,
        ],
    )
    def gather_kernel(idx_hbm, table_hbm, out_hbm, idx_v, rows_v, sem):
        wid = lax.axis_index("s") * NC + lax.axis_index("c")
        base = wid * B_PER_W
        pltpu.sync_copy(idx_hbm.at[wid], idx_v)
        copies = []
        for j in range(NCHUNK):
            copies.append(
                pltpu.async_copy(
                    table_hbm.at[idx_v.at[j]],
                    rows_v.at[pl.ds(j * CHUNK, CHUNK)],
                    sem,
                )
            )
        for c in copies:
            c.wait()
        pltpu.sync_copy(rows_v, out_hbm.at[pl.ds(base, B_PER_W)])

    return gather_kernel


_gather = _make_gather()

BQ = 512
GRID = QC // BQ


def _mlp_body(e_ref, w_ref, b_ref, ws_ref, bs_ref, o_ref):
    acc = jnp.zeros((BQ, PK * HID), jnp.float32)
    for c in range(4):
        h = jnp.dot(e_ref[c], w_ref[c], preferred_element_type=jnp.float32) + b_ref[c]
        acc = acc + jnp.maximum(h, 0.0)
    o_ref[...] = (
        jnp.dot(acc, ws_ref[...], preferred_element_type=jnp.float32) + bs_ref[...]
    )


def kernel(x, table, w1, b1, w2, b2, w3, b3, w4, b4, ws, bs):
    idx = jnp.stack([x[:, 0], x[:, 1], x[:, 3], x[:, 4]], axis=0)
    idx = idx.reshape(NW, NCHUNK, CHUNK)

    rows = _gather(idx, table)
    e_pk = rows.reshape(4, QC, PK * EMB)

    eye = jnp.eye(PK, dtype=jnp.float32)
    w_bd = jnp.stack(
        [jnp.kron(eye, w) for w in (w1, w2, w3, w4)], axis=0
    )
    b_bd = jnp.stack(
        [jnp.tile(b, PK) for b in (b1, b2, b3, b4)], axis=0
    )[:, None, :]
    ws_bd = jnp.kron(eye, ws)
    bs_bd = jnp.tile(bs, PK)[None, :]

    out = pl.pallas_call(
        _mlp_body,
        grid=(GRID,),
        in_specs=[
            pl.BlockSpec((4, BQ, PK * EMB), lambda i: (0, i, 0)),
            pl.BlockSpec((4, PK * EMB, PK * HID), lambda i: (0, 0, 0)),
            pl.BlockSpec((4, 1, PK * HID), lambda i: (0, 0, 0)),
            pl.BlockSpec((PK * HID, PK * EMB), lambda i: (0, 0)),
            pl.BlockSpec((1, PK * EMB), lambda i: (0, 0)),
        ],
        out_specs=pl.BlockSpec((BQ, PK * EMB), lambda i: (i, 0)),
        out_shape=jax.ShapeDtypeStruct((QC, PK * EMB), jnp.float32),
    )(e_pk, w_bd, b_bd, ws_bd, bs_bd)
    return out.reshape(BATCH, EMB)
